# baseline (speedup 1.0000x reference)
import jax
import jax.numpy as jnp
from jax import lax
from jax.experimental import pallas as pl
from jax.experimental.pallas import tpu as pltpu

N_NODES = 100000
D = 129
B = 4096
NC = 20
CB = N_NODES // NC


def _body(sidx_s, perm_s, coff_s, mem_ref, vals_ref, out_ref, csem, rsem):
    for c in range(NC):
        pltpu.make_async_copy(
            mem_ref.at[pl.ds(c * CB, CB)],
            out_ref.at[pl.ds(c * CB, CB)],
            csem.at[c],
        ).start()

    def issue(k, n_issued):
        row = sidx_s[k]
        nxt = sidx_s[jnp.minimum(k + 1, B - 1)]
        is_winner = jnp.logical_or(k == B - 1, row != nxt)

        def do_issue():
            src = perm_s[k]
            pltpu.make_async_copy(
                vals_ref.at[pl.ds(src, 1)],
                out_ref.at[pl.ds(row, 1)],
                rsem,
            ).start()

        pl.when(is_winner)(do_issue)
        return n_issued + is_winner.astype(jnp.int32)

    n_issued = jnp.int32(0)
    for c in range(NC):
        pltpu.make_async_copy(
            mem_ref.at[pl.ds(c * CB, CB)],
            out_ref.at[pl.ds(c * CB, CB)],
            csem.at[c],
        ).wait()
        n_issued = lax.fori_loop(coff_s[c], coff_s[c + 1], issue, n_issued)

    def drain(_, carry):
        pltpu.make_async_copy(
            vals_ref.at[pl.ds(0, 1)],
            out_ref.at[pl.ds(0, 1)],
            rsem,
        ).wait()
        return carry

    lax.fori_loop(0, n_issued, drain, 0)


_call = pl.pallas_call(
    _body,
    grid_spec=pltpu.PrefetchScalarGridSpec(
        num_scalar_prefetch=3,
        grid=(1,),
        in_specs=[
            pl.BlockSpec(memory_space=pltpu.MemorySpace.HBM),
            pl.BlockSpec(memory_space=pltpu.MemorySpace.HBM),
        ],
        out_specs=pl.BlockSpec(memory_space=pltpu.MemorySpace.HBM),
        scratch_shapes=[
            pltpu.SemaphoreType.DMA((NC,)),
            pltpu.SemaphoreType.DMA,
        ],
    ),
    out_shape=jax.ShapeDtypeStruct((N_NODES, D), jnp.float32),
)


def kernel(memory, node_idxs, values):
    idx = node_idxs.astype(jnp.int32)
    perm = jnp.argsort(idx, stable=True).astype(jnp.int32)
    sidx = idx[perm]
    coff = jnp.searchsorted(
        sidx, jnp.arange(0, N_NODES + CB, CB, dtype=jnp.int32)
    ).astype(jnp.int32)
    return _call(sidx, perm, coff, memory, values)


# speedup vs baseline: 11.7002x; 11.7002x over previous
import functools

import jax
import jax.numpy as jnp
from jax import lax
from jax.experimental import pallas as pl
from jax.experimental.pallas import tpu as pltpu

N_NODES = 100000
D = 129
B = 4096
BLK = 1000
GRID = N_NODES // BLK


def _body(sidx_s, perm_s, mem_ref, vals_ref, sidx_v_ref, out_ref):
    i = pl.program_id(0)
    out_ref[...] = mem_ref[...]
    lo = i * BLK
    sidx_v = sidx_v_ref[...]
    cnt_lo = jnp.sum((sidx_v < lo).astype(jnp.int32))
    cnt_hi = jnp.sum((sidx_v < lo + BLK).astype(jnp.int32))

    def apply_one(k, carry):
        row = sidx_s[k] - lo
        src = perm_s[k]
        out_ref[pl.ds(row, 1), :] = vals_ref[pl.ds(src, 1), :]
        return carry

    lax.fori_loop(cnt_lo, cnt_hi, apply_one, 0)


_call = pl.pallas_call(
    _body,
    grid_spec=pltpu.PrefetchScalarGridSpec(
        num_scalar_prefetch=2,
        grid=(GRID,),
        in_specs=[
            pl.BlockSpec((BLK, D), lambda i, *_: (i, 0)),
            pl.BlockSpec((B, D), lambda i, *_: (0, 0)),
            pl.BlockSpec((B,), lambda i, *_: (0,)),
        ],
        out_specs=pl.BlockSpec((BLK, D), lambda i, *_: (i, 0)),
    ),
    out_shape=jax.ShapeDtypeStruct((N_NODES, D), jnp.float32),
    compiler_params=pltpu.CompilerParams(dimension_semantics=("parallel",)),
)


def kernel(memory, node_idxs, values):
    idx = node_idxs.astype(jnp.int32)
    perm = jnp.argsort(idx, stable=True).astype(jnp.int32)
    sidx = idx[perm]
    return _call(sidx, perm, memory, values, sidx)


# speedup vs baseline: 13.1079x; 1.1203x over previous
import functools

import jax
import jax.numpy as jnp
from jax import lax
from jax.experimental import pallas as pl
from jax.experimental.pallas import tpu as pltpu

N_NODES = 100000
D = 129
B = 4096
BLK = 2000
GRID = N_NODES // BLK


def _body(sidx_s, perm_s, mem_ref, vals_ref, sidx_v_ref, out_ref):
    i = pl.program_id(0)
    out_ref[...] = mem_ref[...]
    lo = i * BLK
    sidx_v = sidx_v_ref[...]
    cnt_lo = jnp.sum((sidx_v < lo).astype(jnp.int32))
    cnt_hi = jnp.sum((sidx_v < lo + BLK).astype(jnp.int32))

    def apply_one(k, carry):
        row = sidx_s[k] - lo
        src = perm_s[k]
        out_ref[pl.ds(row, 1), :] = vals_ref[pl.ds(src, 1), :]
        return carry

    lax.fori_loop(cnt_lo, cnt_hi, apply_one, 0)


_call = pl.pallas_call(
    _body,
    grid_spec=pltpu.PrefetchScalarGridSpec(
        num_scalar_prefetch=2,
        grid=(GRID,),
        in_specs=[
            pl.BlockSpec((BLK, D), lambda i, *_: (i, 0)),
            pl.BlockSpec((B, D), lambda i, *_: (0, 0)),
            pl.BlockSpec((B,), lambda i, *_: (0,)),
        ],
        out_specs=pl.BlockSpec((BLK, D), lambda i, *_: (i, 0)),
    ),
    out_shape=jax.ShapeDtypeStruct((N_NODES, D), jnp.float32),
)


def kernel(memory, node_idxs, values):
    idx = node_idxs.astype(jnp.int32)
    perm = jnp.argsort(idx, stable=True).astype(jnp.int32)
    sidx = idx[perm]
    return _call(sidx, perm, memory, values, sidx)


# speedup vs baseline: 13.7380x; 1.0481x over previous
import functools

import jax
import jax.numpy as jnp
from jax import lax
from jax.experimental import pallas as pl
from jax.experimental.pallas import tpu as pltpu

N_NODES = 100000
D = 129
B = 4096
BLK = 4000
GRID = N_NODES // BLK


def _body(sidx_s, perm_s, mem_ref, vals_ref, sidx_v_ref, out_ref):
    i = pl.program_id(0)
    out_ref[...] = mem_ref[...]
    lo = i * BLK
    sidx_v = sidx_v_ref[...]
    cnt_lo = jnp.sum((sidx_v < lo).astype(jnp.int32))
    cnt_hi = jnp.sum((sidx_v < lo + BLK).astype(jnp.int32))

    def apply_one(k, carry):
        row = sidx_s[k] - lo
        src = perm_s[k]
        out_ref[pl.ds(row, 1), :] = vals_ref[pl.ds(src, 1), :]
        return carry

    lax.fori_loop(cnt_lo, cnt_hi, apply_one, 0)


_call = pl.pallas_call(
    _body,
    grid_spec=pltpu.PrefetchScalarGridSpec(
        num_scalar_prefetch=2,
        grid=(GRID,),
        in_specs=[
            pl.BlockSpec((BLK, D), lambda i, *_: (i, 0)),
            pl.BlockSpec((B, D), lambda i, *_: (0, 0)),
            pl.BlockSpec((B,), lambda i, *_: (0,)),
        ],
        out_specs=pl.BlockSpec((BLK, D), lambda i, *_: (i, 0)),
    ),
    out_shape=jax.ShapeDtypeStruct((N_NODES, D), jnp.float32),
)


def kernel(memory, node_idxs, values):
    idx = node_idxs.astype(jnp.int32)
    perm = jnp.argsort(idx, stable=True).astype(jnp.int32)
    sidx = idx[perm]
    return _call(sidx, perm, memory, values, sidx)


# speedup vs baseline: 13.9959x; 1.0188x over previous
import functools

import jax
import jax.numpy as jnp
from jax import lax
from jax.experimental import pallas as pl
from jax.experimental.pallas import tpu as pltpu

N_NODES = 100000
D = 129
B = 4096
BLK = 10000
GRID = N_NODES // BLK


def _body(sidx_s, perm_s, mem_ref, vals_ref, sidx_v_ref, out_ref):
    i = pl.program_id(0)
    out_ref[...] = mem_ref[...]
    lo = i * BLK
    sidx_v = sidx_v_ref[...]
    cnt_lo = jnp.sum((sidx_v < lo).astype(jnp.int32))
    cnt_hi = jnp.sum((sidx_v < lo + BLK).astype(jnp.int32))

    def apply_one(k, carry):
        row = sidx_s[k] - lo
        src = perm_s[k]
        out_ref[pl.ds(row, 1), :] = vals_ref[pl.ds(src, 1), :]
        return carry

    lax.fori_loop(cnt_lo, cnt_hi, apply_one, 0)


_call = pl.pallas_call(
    _body,
    grid_spec=pltpu.PrefetchScalarGridSpec(
        num_scalar_prefetch=2,
        grid=(GRID,),
        in_specs=[
            pl.BlockSpec((BLK, D), lambda i, *_: (i, 0)),
            pl.BlockSpec((B, D), lambda i, *_: (0, 0)),
            pl.BlockSpec((B,), lambda i, *_: (0,)),
        ],
        out_specs=pl.BlockSpec((BLK, D), lambda i, *_: (i, 0)),
    ),
    out_shape=jax.ShapeDtypeStruct((N_NODES, D), jnp.float32),
)


def kernel(memory, node_idxs, values):
    idx = node_idxs.astype(jnp.int32)
    perm = jnp.argsort(idx, stable=True).astype(jnp.int32)
    sidx = idx[perm]
    return _call(sidx, perm, memory, values, sidx)
